# Initial kernel scaffold; baseline (speedup 1.0000x reference)
import jax
import jax.numpy as jnp
from jax.experimental import pallas as pl


def kernel(x, indices, values):
    raise NotImplementedError("write your pallas kernel here")


# speedup vs baseline: 2.2761x; 2.2761x over previous
import functools

import jax
import jax.numpy as jnp
from jax import lax
from jax.experimental import pallas as pl
from jax.experimental.pallas import tpu as pltpu
from jax.experimental.pallas import tpu_sc as plsc

S = 64
K = 256
L = 16


def _spmv_body(x_hbm, rows_hbm, cols_hbm, vals_hbm, out_hbm,
               x_v, rows_v, cols_v, vals_v, acc_v, sem):
    wid = lax.axis_index("s") * 2 + lax.axis_index("c")

    @pl.when(wid == 0)
    def _():
        cp_x = pltpu.make_async_copy(x_hbm, x_v, sem)
        cp_r = pltpu.make_async_copy(rows_hbm, rows_v, sem)
        cp_c = pltpu.make_async_copy(cols_hbm, cols_v, sem)
        cp_v = pltpu.make_async_copy(vals_hbm, vals_v, sem)
        cp_x.start()
        cp_r.start()
        cp_c.start()
        cp_v.start()
        cp_x.wait()
        cp_r.wait()
        cp_c.wait()
        cp_v.wait()

        zero = jnp.zeros((L,), jnp.float32)
        for j in range(S // L):
            acc_v[pl.ds(j * L, L)] = zero

        for i in range(K // L):
            r = rows_v[pl.ds(i * L, L)]
            c = cols_v[pl.ds(i * L, L)]
            v = vals_v[pl.ds(i * L, L)]
            g = plsc.load_gather(x_v, [c])
            plsc.addupdate_scatter(acc_v, [r], v * g)

        pltpu.sync_copy(acc_v, out_hbm)


@jax.jit
def _spmv(x, rows, cols, vals):
    mesh = plsc.VectorSubcoreMesh(core_axis_name="c", subcore_axis_name="s")
    return pl.kernel(
        _spmv_body,
        out_type=jax.ShapeDtypeStruct((S,), jnp.float32),
        mesh=mesh,
        scratch_types=[
            pltpu.VMEM((S,), jnp.float32),
            pltpu.VMEM((K,), jnp.int32),
            pltpu.VMEM((K,), jnp.int32),
            pltpu.VMEM((K,), jnp.float32),
            pltpu.VMEM((S,), jnp.float32),
            pltpu.SemaphoreType.DMA,
        ],
        compiler_params=pltpu.CompilerParams(needs_layout_passes=False),
    )(x, rows, cols, vals)


def kernel(x, indices, values):
    rows = indices[0].astype(jnp.int32)
    cols = indices[1].astype(jnp.int32)
    return _spmv(x, rows, cols, values)


# speedup vs baseline: 2.4531x; 1.0778x over previous
import functools

import jax
import jax.numpy as jnp
from jax import lax
from jax.experimental import pallas as pl
from jax.experimental.pallas import tpu as pltpu
from jax.experimental.pallas import tpu_sc as plsc

S = 64
K = 256
L = 16


def _spmv_body(x_hbm, idx_hbm, vals_hbm, out_hbm,
               x_v, idx_v, vals_v, acc_v, sem):
    sid = lax.axis_index("s")

    @pl.when(sid == 0)
    def _():
        cp_x = pltpu.make_async_copy(x_hbm, x_v, sem)
        cp_i = pltpu.make_async_copy(idx_hbm, idx_v, sem)
        cp_v = pltpu.make_async_copy(vals_hbm, vals_v, sem)
        cp_x.start()
        cp_i.start()
        cp_v.start()
        cp_x.wait()
        cp_i.wait()
        cp_v.wait()

        zero = jnp.zeros((L,), jnp.float32)
        for j in range(S // L):
            acc_v[pl.ds(j * L, L)] = zero

        for i in range(K // L):
            r = idx_v[0, pl.ds(i * L, L)]
            c = idx_v[1, pl.ds(i * L, L)]
            v = vals_v[pl.ds(i * L, L)]
            g = plsc.load_gather(x_v, [c])
            plsc.addupdate_scatter(acc_v, [r], v * g)

        pltpu.sync_copy(acc_v, out_hbm)


@jax.jit
def _spmv(x, idx, vals):
    mesh = plsc.VectorSubcoreMesh(
        core_axis_name="c", subcore_axis_name="s", num_cores=1)
    return pl.kernel(
        _spmv_body,
        out_type=jax.ShapeDtypeStruct((S,), jnp.float32),
        mesh=mesh,
        scratch_types=[
            pltpu.VMEM((S,), jnp.float32),
            pltpu.VMEM((2, K), jnp.int32),
            pltpu.VMEM((K,), jnp.float32),
            pltpu.VMEM((S,), jnp.float32),
            pltpu.SemaphoreType.DMA,
        ],
        compiler_params=pltpu.CompilerParams(needs_layout_passes=False),
    )(x, idx, vals)


def kernel(x, indices, values):
    return _spmv(x, indices.astype(jnp.int32), values)
